# baseline (speedup 1.0000x reference)
import jax
import jax.numpy as jnp
from jax.experimental import pallas as pl

_N_LAYERS = 3
_D_H = 128
_D_MEM = 64


def _dot(a, b):
    return jax.lax.dot_general(a, b, (((1,), (0,)), ((), ())),
                               preferred_element_type=jnp.float32)


def _dot_t(a, b):
    return jax.lax.dot_general(a, b, (((1,), (1,)), ((), ())),
                               preferred_element_type=jnp.float32)


def _fused_body(x_ref, a128_ref, a192_ref, a64_ref, bias_ref, out_ref):
    x = x_ref[...]
    A = a128_ref[...]
    B = a192_ref[...]
    C = a64_ref[...]
    D = bias_ref[...]

    proj_W = A[0:128]
    cls_W1 = A[1472:1536]
    cls_W2 = C[576:577]
    mem = C[584:585]
    proj_b = D[0:1, 0:128]
    cls_b1 = D[128:129, 0:64]

    a = x
    Mt = proj_W
    c = proj_b
    hbar = _dot_t(jnp.mean(x, axis=0, keepdims=True), Mt) + c
    for l in range(_N_LAYERS):
        base = 128 + 448 * l
        Wih = A[base:base + 192]
        msg_W2 = A[base + 192:base + 320]
        agg_W = A[base + 320:base + 448]
        msg_W1 = B[128 * l:128 * l + 128]
        Whh = C[192 * l:192 * l + 192]
        brow = 8 + 40 * l
        bih = D[brow:brow + 1]
        bhh = D[brow + 8:brow + 9]
        msg_b1 = D[brow + 16:brow + 17, 0:128]
        msg_b2 = D[brow + 24:brow + 25, 0:128]
        agg_b = D[brow + 32:brow + 33, 0:128]

        gi = _dot_t(hbar, Wih) + bih
        gh = _dot_t(mem, Whh) + bhh
        r = jax.nn.sigmoid(gi[:, 0:_D_MEM] + gh[:, 0:_D_MEM])
        z = jax.nn.sigmoid(gi[:, _D_MEM:2 * _D_MEM] + gh[:, _D_MEM:2 * _D_MEM])
        n = jnp.tanh(gi[:, 2 * _D_MEM:] + r * gh[:, 2 * _D_MEM:])
        mem = (1.0 - z) * n + z * mem

        W1h = msg_W1[:, :_D_H]
        mvec = _dot_t(mem, msg_W1[:, _D_H:]) + msg_b1
        G = _dot(W1h, Mt)
        g = _dot_t(c, W1h) + mvec
        a = jax.nn.relu(_dot_t(a, G) + g)
        Mt = _dot(agg_W, msg_W2)
        c = _dot_t(msg_b2, agg_W) + agg_b
        if l + 1 < _N_LAYERS:
            hbar = _dot_t(jnp.mean(a, axis=0, keepdims=True), Mt) + c

    Gc = _dot(cls_W1, Mt)
    gc = _dot_t(c, cls_W1) + cls_b1
    c1 = jax.nn.relu(_dot_t(a, Gc) + gc)
    out_ref[...] = _dot_t(c1, cls_W2)


def kernel(x, edge_index, edge_attr, edge_time, params):
    p = params
    ls = p['layers']

    a128_rows = [p['proj_W']]
    for lp in ls:
        a128_rows += [lp['Wih'], lp['msg_W2'], lp['agg_W']]
    a128_rows.append(p['cls_W1'])
    a128 = jnp.concatenate(a128_rows, axis=0)

    a192 = jnp.concatenate([lp['msg_W1'] for lp in ls], axis=0)

    z7 = jnp.zeros((7, _D_MEM), jnp.float32)
    a64 = jnp.concatenate(
        [lp['Whh'] for lp in ls]
        + [p['cls_W2'], z7, p['memory'], z7], axis=0)

    def brow(b):
        r = jnp.zeros((8, 192), jnp.float32)
        return r.at[0, :b.shape[0]].set(b)

    bias_rows = [brow(p['proj_b'])]
    for lp in ls:
        bias_rows += [brow(lp['bih']), brow(lp['bhh']), brow(lp['msg_b1']),
                      brow(lp['msg_b2']), brow(lp['agg_b'])]
    bias_rows.append(brow(p['cls_b1']))
    bias = jnp.concatenate(bias_rows, axis=0)

    out = pl.pallas_call(
        _fused_body,
        out_shape=jax.ShapeDtypeStruct((x.shape[0], 1), jnp.float32),
    )(x, a128, a192, a64, bias)
    return out + p['cls_b2']
